# baseline (speedup 1.0000x reference)
import jax
import jax.numpy as jnp
from jax import lax
from jax.experimental import pallas as pl
from jax.experimental.pallas import tpu as pltpu

_BI = 512


def _rgcn_body(X_ref, A_ref, Wl_ref, Win_ref, Wout_ref, out_ref,
               xT, hin_aug, houtT_aug, y2t_aug):
    r = pl.program_id(0)
    i = pl.program_id(1)
    ni = pl.num_programs(1)
    n, od = out_ref.shape
    bi = A_ref.shape[1]
    aug_w = hin_aug.shape[1]
    aug_h = houtT_aug.shape[0]

    @pl.when(jnp.logical_and(r == 0, i == 0))
    def _init():
        out_ref[...] = jnp.dot(X_ref[...], Wl_ref[...],
                               preferred_element_type=jnp.float32)
        xT[...] = X_ref[...].T
        cid = lax.broadcasted_iota(jnp.int32, (n, aug_w - od), 1)
        hin_aug[:, od:] = jnp.where(cid == 0, 1.0, 0.0).astype(jnp.bfloat16)
        rid = lax.broadcasted_iota(jnp.int32, (aug_h - od, n), 0)
        houtT_aug[od:, :] = jnp.where(rid == 0, 1.0, 0.0).astype(jnp.bfloat16)

    @pl.when(i == 0)
    def _per_relation():
        hin_aug[:, :od] = jnp.dot(
            X_ref[...], Win_ref[0],
            preferred_element_type=jnp.float32).astype(jnp.bfloat16)
        houtT_aug[:od, :] = lax.dot_general(
            Wout_ref[0], xT[...], (((0,), (0,)), ((), ())),
            preferred_element_type=jnp.float32).astype(jnp.bfloat16)
        y2t_aug[...] = jnp.zeros_like(y2t_aug)

    a = A_ref[0].astype(jnp.bfloat16)
    p = jnp.dot(a, hin_aug[...], preferred_element_type=jnp.float32)
    inv_r = 1.0 / jnp.maximum(p[:, od:od + 1], 1e-12)
    out_ref[pl.ds(i * bi, bi), :] += p[:, :od] * inv_r
    y2t_aug[...] += jnp.dot(houtT_aug[:, pl.ds(i * bi, bi)], a,
                            preferred_element_type=jnp.float32)

    @pl.when(i == ni - 1)
    def _finish_relation():
        inv_c = 1.0 / jnp.maximum(y2t_aug[od:od + 1, :], 1e-12)
        out_ref[...] += (y2t_aug[:od, :] * inv_c).T


def kernel(X, A, W_loop, W_in, W_out):
    n, in_dim = X.shape
    r_count = A.shape[0]
    out_dim = W_loop.shape[1]
    bi = min(_BI, n)
    ni = n // bi
    aug_w = max(128, out_dim + 1)
    aug_h = out_dim + 8

    return pl.pallas_call(
        _rgcn_body,
        grid=(r_count, ni),
        in_specs=[
            pl.BlockSpec((n, in_dim), lambda r, i: (0, 0)),
            pl.BlockSpec((1, bi, n), lambda r, i: (r, i, 0)),
            pl.BlockSpec((in_dim, out_dim), lambda r, i: (0, 0)),
            pl.BlockSpec((1, in_dim, out_dim), lambda r, i: (r, 0, 0)),
            pl.BlockSpec((1, in_dim, out_dim), lambda r, i: (r, 0, 0)),
        ],
        out_specs=pl.BlockSpec((n, out_dim), lambda r, i: (0, 0)),
        out_shape=jax.ShapeDtypeStruct((n, out_dim), jnp.float32),
        scratch_shapes=[
            pltpu.VMEM((in_dim, n), jnp.float32),
            pltpu.VMEM((n, aug_w), jnp.bfloat16),
            pltpu.VMEM((aug_h, n), jnp.bfloat16),
            pltpu.VMEM((aug_h, n), jnp.float32),
        ],
    )(X, A, W_loop, W_in, W_out)


# speedup vs baseline: 1.0824x; 1.0824x over previous
import jax
import jax.numpy as jnp
from jax import lax
from jax.experimental import pallas as pl
from jax.experimental.pallas import tpu as pltpu

_BI = 1024


def _rgcn_body(X_ref, A_ref, Wl_ref, Win_ref, Wout_ref, out_ref,
               xT, hin_aug, houtT_aug, y2t_a, y2t_b):
    r = pl.program_id(0)
    i = pl.program_id(1)
    nr = pl.num_programs(0)
    ni = pl.num_programs(1)
    n, od = out_ref.shape
    bi = A_ref.shape[1]
    aug_w = hin_aug.shape[1]
    aug_h = houtT_aug.shape[0]
    even = r % 2 == 0
    first = i == 0

    @pl.when(jnp.logical_and(r == 0, first))
    def _init():
        out_ref[...] = jnp.dot(X_ref[...], Wl_ref[...],
                               preferred_element_type=jnp.float32)
        xT[...] = X_ref[...].T
        cid = lax.broadcasted_iota(jnp.int32, (n, aug_w - od), 1)
        hin_aug[:, od:] = jnp.where(cid == 0, 1.0, 0.0).astype(jnp.bfloat16)
        rid = lax.broadcasted_iota(jnp.int32, (aug_h - od, n), 0)
        houtT_aug[od:, :] = jnp.where(rid == 0, 1.0, 0.0).astype(jnp.bfloat16)

    @pl.when(first)
    def _per_relation():
        hin_aug[:, :od] = jnp.dot(
            X_ref[...], Win_ref[0],
            preferred_element_type=jnp.float32).astype(jnp.bfloat16)
        houtT_aug[:od, :] = lax.dot_general(
            Wout_ref[0], xT[...], (((0,), (0,)), ((), ())),
            preferred_element_type=jnp.float32).astype(jnp.bfloat16)

    def _finish(buf):
        inv_c = 1.0 / jnp.maximum(buf[od:od + 1, :], 1e-12)
        out_ref[...] += (buf[:od, :] * inv_c).T

    @pl.when(jnp.logical_and(r > 0, jnp.logical_and(i == 1, jnp.logical_not(even))))
    def _finish_prev_from_a():
        _finish(y2t_a[...])

    @pl.when(jnp.logical_and(r > 0, jnp.logical_and(i == 1, even)))
    def _finish_prev_from_b():
        _finish(y2t_b[...])

    a = A_ref[0].astype(jnp.bfloat16)
    p = jnp.dot(a, hin_aug[...], preferred_element_type=jnp.float32)
    inv_r = 1.0 / jnp.maximum(p[:, od:od + 1], 1e-12)
    out_ref[pl.ds(i * bi, bi), :] += p[:, :od] * inv_r
    contrib = jnp.dot(houtT_aug[:, pl.ds(i * bi, bi)], a,
                      preferred_element_type=jnp.float32)

    @pl.when(jnp.logical_and(even, first))
    def _store_a():
        y2t_a[...] = contrib

    @pl.when(jnp.logical_and(even, jnp.logical_not(first)))
    def _accum_a():
        y2t_a[...] += contrib

    @pl.when(jnp.logical_and(jnp.logical_not(even), first))
    def _store_b():
        y2t_b[...] = contrib

    @pl.when(jnp.logical_and(jnp.logical_not(even), jnp.logical_not(first)))
    def _accum_b():
        y2t_b[...] += contrib

    @pl.when(jnp.logical_and(r == nr - 1, i == ni - 1))
    def _finish_last():
        @pl.when(jnp.logical_not(even))
        def _from_b():
            _finish(y2t_b[...])

        @pl.when(even)
        def _from_a():
            _finish(y2t_a[...])


def kernel(X, A, W_loop, W_in, W_out):
    n, in_dim = X.shape
    r_count = A.shape[0]
    out_dim = W_loop.shape[1]
    bi = min(_BI, n)
    ni = n // bi
    aug_w = max(128, out_dim + 1)
    aug_h = out_dim + 8

    return pl.pallas_call(
        _rgcn_body,
        grid=(r_count, ni),
        in_specs=[
            pl.BlockSpec((n, in_dim), lambda r, i: (0, 0)),
            pl.BlockSpec((1, bi, n), lambda r, i: (r, i, 0)),
            pl.BlockSpec((in_dim, out_dim), lambda r, i: (0, 0)),
            pl.BlockSpec((1, in_dim, out_dim), lambda r, i: (r, 0, 0)),
            pl.BlockSpec((1, in_dim, out_dim), lambda r, i: (r, 0, 0)),
        ],
        out_specs=pl.BlockSpec((n, out_dim), lambda r, i: (0, 0)),
        out_shape=jax.ShapeDtypeStruct((n, out_dim), jnp.float32),
        scratch_shapes=[
            pltpu.VMEM((in_dim, n), jnp.float32),
            pltpu.VMEM((n, aug_w), jnp.bfloat16),
            pltpu.VMEM((aug_h, n), jnp.bfloat16),
            pltpu.VMEM((aug_h, n), jnp.float32),
            pltpu.VMEM((aug_h, n), jnp.float32),
        ],
    )(X, A, W_loop, W_in, W_out)


# speedup vs baseline: 1.1057x; 1.0215x over previous
import jax
import jax.numpy as jnp
from jax import lax
from jax.experimental import pallas as pl
from jax.experimental.pallas import tpu as pltpu

_BI = 1024


def _rgcn_body(X_ref, A_ref, Wl_ref, Win_ref, Wout_ref, out_ref,
               xT, hin_aug, houtT_aug, y2t_aug):
    r = pl.program_id(0)
    i = pl.program_id(1)
    ni = pl.num_programs(1)
    n, od = out_ref.shape
    bi = A_ref.shape[1]
    aug_w = hin_aug.shape[1]
    aug_h = houtT_aug.shape[0]

    @pl.when(jnp.logical_and(r == 0, i == 0))
    def _init():
        out_ref[...] = jnp.dot(X_ref[...], Wl_ref[...],
                               preferred_element_type=jnp.float32)
        xT[...] = X_ref[...].T
        cid = lax.broadcasted_iota(jnp.int32, (n, aug_w - od), 1)
        hin_aug[:, od:] = jnp.where(cid == 0, 1.0, 0.0).astype(jnp.bfloat16)
        rid = lax.broadcasted_iota(jnp.int32, (aug_h - od, n), 0)
        houtT_aug[od:, :] = jnp.where(rid == 0, 1.0, 0.0).astype(jnp.bfloat16)

    @pl.when(i == 0)
    def _per_relation():
        hin_aug[:, :od] = jnp.dot(
            X_ref[...], Win_ref[0],
            preferred_element_type=jnp.float32).astype(jnp.bfloat16)
        houtT_aug[:od, :] = lax.dot_general(
            Wout_ref[0], xT[...], (((0,), (0,)), ((), ())),
            preferred_element_type=jnp.float32).astype(jnp.bfloat16)
        y2t_aug[...] = jnp.zeros_like(y2t_aug)

    a = A_ref[0].astype(jnp.bfloat16)
    p = jnp.dot(a, hin_aug[...], preferred_element_type=jnp.float32)
    inv_r = 1.0 / jnp.maximum(p[:, od:od + 1], 1e-12)
    out_ref[pl.ds(i * bi, bi), :] += p[:, :od] * inv_r
    y2t_aug[...] += jnp.dot(houtT_aug[:, pl.ds(i * bi, bi)], a,
                            preferred_element_type=jnp.float32)

    @pl.when(i == ni - 1)
    def _finish_relation():
        inv_c = 1.0 / jnp.maximum(y2t_aug[od:od + 1, :], 1e-12)
        out_ref[...] += (y2t_aug[:od, :] * inv_c).T


def kernel(X, A, W_loop, W_in, W_out):
    n, in_dim = X.shape
    r_count = A.shape[0]
    out_dim = W_loop.shape[1]
    bi = min(_BI, n)
    ni = n // bi
    aug_w = max(128, out_dim + 1)
    aug_h = out_dim + 8

    return pl.pallas_call(
        _rgcn_body,
        grid=(r_count, ni),
        in_specs=[
            pl.BlockSpec((n, in_dim), lambda r, i: (0, 0)),
            pl.BlockSpec((1, bi, n), lambda r, i: (r, i, 0)),
            pl.BlockSpec((in_dim, out_dim), lambda r, i: (0, 0)),
            pl.BlockSpec((1, in_dim, out_dim), lambda r, i: (r, 0, 0)),
            pl.BlockSpec((1, in_dim, out_dim), lambda r, i: (r, 0, 0)),
        ],
        out_specs=pl.BlockSpec((n, out_dim), lambda r, i: (0, 0)),
        out_shape=jax.ShapeDtypeStruct((n, out_dim), jnp.float32),
        scratch_shapes=[
            pltpu.VMEM((in_dim, n), jnp.float32),
            pltpu.VMEM((n, aug_w), jnp.bfloat16),
            pltpu.VMEM((aug_h, n), jnp.bfloat16),
            pltpu.VMEM((aug_h, n), jnp.float32),
        ],
    )(X, A, W_loop, W_in, W_out)
